# baseline (speedup 1.0000x reference)
import functools

import jax
import jax.numpy as jnp
from jax import lax
from jax.experimental import pallas as pl
from jax.experimental.pallas import tpu as pltpu
from jax.experimental.pallas import tpu_sc as plsc

NUM_ROWS = 16384
NUM_COLS = 26
NUM_EMB = 1000000
DIM = 64
PCOLS = 32
PDIM = 128

_info = plsc.get_sparse_core_info()
NC = _info.num_cores
NS = _info.num_subcores
NW = NC * NS
ROWS_PER_W = NUM_ROWS // NW
B_PER_W = ROWS_PER_W * NUM_COLS
CROWS = 8
N_CHUNKS = ROWS_PER_W // CROWS
NBUF = 2
LANES = 16

assert N_CHUNKS * CROWS == ROWS_PER_W
assert N_CHUNKS % 2 == 0


def _body(x_hbm, w_hbm, out_hbm, idx2d_v, idx_v, rows_v, *sems):
    gsem = sems[:NBUF]
    osem = sems[NBUF:]
    wid = lax.axis_index("s") * NC + lax.axis_index("c")
    row0 = wid * ROWS_PER_W

    pltpu.sync_copy(x_hbm.at[pl.ds(row0, ROWS_PER_W)], idx2d_v)

    lane = lax.broadcasted_iota(jnp.int32, (LANES,), 0)
    zeros = jnp.zeros((LANES,), jnp.int32)

    def zero_step(t, _):
        idx_v[t // 2, pl.ds((t % 2) * LANES, LANES)] = zeros
        return ()

    lax.fori_loop(0, ROWS_PER_W * PCOLS // LANES, zero_step, ())

    def xform_step(t, _):
        p = t * LANES + lane
        r = p // NUM_COLS
        c = p % NUM_COLS
        v = plsc.load_gather(idx2d_v, [r, c])
        plsc.store_scatter(idx_v, [r, c], v)
        return ()

    lax.fori_loop(0, B_PER_W // LANES, xform_step, ())

    def fire_gather(g, b):
        for i in range(CROWS):
            pltpu.async_copy(
                w_hbm.at[idx_v.at[g * CROWS + i]],
                rows_v.at[b, i],
                gsem[b],
            )

    def wait_gather(b):
        for i in range(CROWS):
            pltpu.make_async_copy(
                w_hbm.at[idx_v.at[0]],
                rows_v.at[b, i],
                gsem[b],
            ).wait()

    def fire_out(g, b):
        pltpu.async_copy(
            rows_v.at[b], out_hbm.at[pl.ds(row0 + g * CROWS, CROWS)], osem[b]
        )

    def wait_out(b):
        pltpu.make_async_copy(
            rows_v.at[b], out_hbm.at[pl.ds(row0, CROWS)], osem[b]
        ).wait()

    fire_gather(0, 0)

    wait_gather(0)
    fire_out(0, 0)
    fire_gather(1, 1)

    def pair(k, _):
        g = 2 * k + 1
        wait_gather(1)
        fire_out(g, 1)
        wait_out(0)
        fire_gather(g + 1, 0)
        wait_gather(0)
        fire_out(g + 1, 0)
        wait_out(1)
        fire_gather(g + 2, 1)
        return ()

    lax.fori_loop(0, (N_CHUNKS - 2) // 2, pair, ())

    wait_gather(1)
    fire_out(N_CHUNKS - 1, 1)
    wait_out(0)
    wait_out(1)


def kernel(x, weight):
    wp = jnp.pad(
        weight.reshape(NUM_EMB // 8, 8, DIM), ((0, 0), (0, 0), (0, PDIM - DIM))
    ).reshape(NUM_EMB, PDIM)
    mesh = plsc.VectorSubcoreMesh(core_axis_name="c", subcore_axis_name="s")
    run = functools.partial(
        pl.kernel,
        mesh=mesh,
        out_type=jax.ShapeDtypeStruct((NUM_ROWS, PCOLS, PDIM), jnp.float32),
        scratch_types=[
            pltpu.VMEM((ROWS_PER_W, NUM_COLS), jnp.int32),
            pltpu.VMEM((ROWS_PER_W, PCOLS), jnp.int32),
            pltpu.VMEM((NBUF, CROWS, PCOLS, PDIM), jnp.float32),
        ]
        + [pltpu.SemaphoreType.DMA] * (2 * NBUF),
        compiler_params=pltpu.CompilerParams(
            use_tc_tiling_on_sc=False, needs_layout_passes=False
        ),
    )(_body)
    out5 = run(x, wp)
    return out5[:, :NUM_COLS, :DIM]
